# Initial kernel scaffold; baseline (speedup 1.0000x reference)
import jax
import jax.numpy as jnp
from jax.experimental import pallas as pl


def kernel(x, ln1_g, ln1_b, wq1, wk1, wv1, wo1, gate_w, fc1_w, fc1_b, fc2_w, fc2_b, ln2_g, ln2_b, wq2, wk2, wv2, wo2):
    raise NotImplementedError("write your pallas kernel here")


# speedup vs baseline: 1.7947x; 1.7947x over previous
import functools

import jax
import jax.numpy as jnp
from jax.experimental import pallas as pl
from jax.experimental.pallas import tpu as pltpu

D = 1024
E = 8
H = 32
HD = D // H
F = 4 * D

TN = 512
TQ = 256
TF = 1024


def _qkv_body(x_ref, g_ref, b_ref, wq_ref, wk_ref, wv_ref, q_ref, k_ref, v_ref):
    x = x_ref[0]
    m = jnp.mean(x, axis=-1, keepdims=True)
    v = jnp.mean((x - m) ** 2, axis=-1, keepdims=True)
    h = (x - m) * jax.lax.rsqrt(v + 1e-5) * g_ref[0] + b_ref[0]
    q_ref[0] = jnp.dot(h, wq_ref[...], preferred_element_type=jnp.float32)
    k_ref[0] = jnp.dot(h, wk_ref[...], preferred_element_type=jnp.float32)
    v_ref[0] = jnp.dot(h, wv_ref[...], preferred_element_type=jnp.float32)


def _attn_core_body(q_ref, k_ref, v_ref, x_ref, wo_ref, o_ref, acc_ref):
    scale = 1.0 / float(HD) ** 0.5
    for h in range(H):
        sl = slice(h * HD, (h + 1) * HD)
        qh = q_ref[0, :, sl]
        kh = k_ref[0, :, sl]
        vh = v_ref[0, :, sl]
        s = jax.lax.dot_general(qh, kh, (((1,), (1,)), ((), ())),
                                preferred_element_type=jnp.float32) * scale
        mx = jnp.max(s, axis=-1, keepdims=True)
        e = jnp.exp(s - mx)
        p = e / jnp.sum(e, axis=-1, keepdims=True)
        acc_ref[:, sl] = jax.lax.dot_general(p, vh, (((1,), (0,)), ((), ())),
                                             preferred_element_type=jnp.float32)
    o_ref[0] = (jnp.dot(acc_ref[...], wo_ref[...],
                        preferred_element_type=jnp.float32) + x_ref[0])


def _attn(x, g, b, wq, wk, wv, wo):
    B, S, _ = x.shape
    g2 = g.reshape(1, D)
    b2 = b.reshape(1, D)
    qkv = pl.pallas_call(
        _qkv_body,
        grid=(B, S // TN),
        in_specs=[
            pl.BlockSpec((1, TN, D), lambda bb, i: (bb, i, 0)),
            pl.BlockSpec((1, D), lambda bb, i: (0, 0)),
            pl.BlockSpec((1, D), lambda bb, i: (0, 0)),
            pl.BlockSpec((D, D), lambda bb, i: (0, 0)),
            pl.BlockSpec((D, D), lambda bb, i: (0, 0)),
            pl.BlockSpec((D, D), lambda bb, i: (0, 0)),
        ],
        out_specs=[pl.BlockSpec((1, TN, D), lambda bb, i: (bb, i, 0))] * 3,
        out_shape=[jax.ShapeDtypeStruct((B, S, D), jnp.float32)] * 3,
    )
    q, k, v = qkv(x, g2, b2, wq, wk, wv)

    out = pl.pallas_call(
        _attn_core_body,
        grid=(B, S // TQ),
        in_specs=[
            pl.BlockSpec((1, TQ, D), lambda bb, i: (bb, i, 0)),
            pl.BlockSpec((1, S, D), lambda bb, i: (bb, 0, 0)),
            pl.BlockSpec((1, S, D), lambda bb, i: (bb, 0, 0)),
            pl.BlockSpec((1, TQ, D), lambda bb, i: (bb, i, 0)),
            pl.BlockSpec((D, D), lambda bb, i: (0, 0)),
        ],
        out_specs=pl.BlockSpec((1, TQ, D), lambda bb, i: (bb, i, 0)),
        out_shape=jax.ShapeDtypeStruct((B, S, D), jnp.float32),
        scratch_shapes=[pltpu.VMEM((TQ, D), jnp.float32)],
    )
    return out(q, k, v, x, wo)


def _ffn_body(h_ref, w1_ref, b1_ref, w2_ref, b2_ref, o_ref):
    f = pl.program_id(1)
    B = h_ref.shape[0]
    rows = B * h_ref.shape[2]
    h = h_ref[...].reshape(rows, D)
    mid = jnp.dot(h, w1_ref[0], preferred_element_type=jnp.float32) + b1_ref[0, 0]
    mid = 0.5 * mid * (1.0 + jax.lax.erf(mid * 0.7071067811865476))
    part = jnp.dot(mid, w2_ref[0], preferred_element_type=jnp.float32)

    @pl.when(f == 0)
    def _():
        o_ref[...] = jnp.broadcast_to(b2_ref[0, 0], (rows, D)).reshape(o_ref.shape)

    o_ref[...] += part.reshape(o_ref.shape)


def _ffn(h, fc1_w, fc1_b, fc2_w, fc2_b):
    B, S, _ = h.shape
    SE = S // E
    h4 = h.reshape(B, E, SE, D)
    b1 = fc1_b.reshape(E, 1, F)
    b2 = fc2_b.reshape(E, 1, D)
    out = pl.pallas_call(
        _ffn_body,
        grid=(E, F // TF),
        in_specs=[
            pl.BlockSpec((B, 1, SE, D), lambda e, f: (0, e, 0, 0)),
            pl.BlockSpec((1, D, TF), lambda e, f: (e, 0, f)),
            pl.BlockSpec((1, 1, TF), lambda e, f: (e, 0, f)),
            pl.BlockSpec((1, TF, D), lambda e, f: (e, f, 0)),
            pl.BlockSpec((1, 1, D), lambda e, f: (e, 0, 0)),
        ],
        out_specs=pl.BlockSpec((B, 1, SE, D), lambda e, f: (0, e, 0, 0)),
        out_shape=jax.ShapeDtypeStruct((B, E, SE, D), jnp.float32),
    )
    return out(h4, fc1_w, b1, fc2_w, b2).reshape(B, S, D)


def kernel(x, ln1_g, ln1_b, wq1, wk1, wv1, wo1, gate_w, fc1_w, fc1_b, fc2_w,
           fc2_b, ln2_g, ln2_b, wq2, wk2, wv2, wo2):
    h1 = _attn(x, ln1_g, ln1_b, wq1, wk1, wv1, wo1)
    eo = _ffn(h1, fc1_w, fc1_b, fc2_w, fc2_b)
    return _attn(eo, ln2_g, ln2_b, wq2, wk2, wv2, wo2)
